# Initial kernel scaffold; baseline (speedup 1.0000x reference)
import jax
import jax.numpy as jnp
from jax.experimental import pallas as pl


def kernel(features, Wc, gamma, beta, memory_bank, perm):
    raise NotImplementedError("write your pallas kernel here")


# speedup vs baseline: 21.4553x; 21.4553x over previous
import functools

import jax
import jax.numpy as jnp
from jax.experimental import pallas as pl
from jax.experimental.pallas import tpu as pltpu

_K = 64
_NCLUS = 8
_TEMP = 0.1
_BN_EPS = 1e-5
_TQ = 256


def _proj_kernel(f_ref, wc_ref, gamma_ref, beta_ref, mtail_ref,
                 feats_ref, qn_ref, mn_ref):
    x = jax.lax.dot_general(f_ref[...], wc_ref[...], (((1,), (1,)), ((), ())),
                            preferred_element_type=jnp.float32)
    mean = jnp.mean(x, axis=0, keepdims=True)
    var = jnp.mean((x - mean) ** 2, axis=0, keepdims=True)
    xn = (x - mean) / jnp.sqrt(var + _BN_EPS)
    feats = jnp.maximum(xn * gamma_ref[...] + beta_ref[...], 0.0)
    feats_ref[...] = feats

    qn = jnp.sum(feats * feats, axis=1, keepdims=True)
    qn_ref[...] = qn

    ones = jnp.ones((1, feats.shape[1]), jnp.float32)
    mn_q = jax.lax.dot_general(ones, feats * feats, (((1,), (1,)), ((), ())),
                               preferred_element_type=jnp.float32)
    mtail = mtail_ref[...]
    mn_t = jax.lax.dot_general(ones, mtail * mtail, (((1,), (1,)), ((), ())),
                               preferred_element_type=jnp.float32)
    mn_ref[...] = jnp.concatenate([mn_q, mn_t], axis=1)


def _density_kernel(q_ref, qn_ref, m_ref, mn_ref, dens_ref):
    prod = jax.lax.dot_general(q_ref[...], m_ref[...], (((1,), (1,)), ((), ())),
                               preferred_element_type=jnp.float32)
    d2 = jnp.maximum(qn_ref[...] + mn_ref[...] - 2.0 * prod, 1e-12)

    lo = jax.lax.bitcast_convert_type(jnp.min(d2, axis=1, keepdims=True),
                                      jnp.int32)
    hi = jax.lax.bitcast_convert_type(jnp.max(d2, axis=1, keepdims=True),
                                      jnp.int32)

    def body(_, carry):
        lo, hi = carry
        mid = lo + ((hi - lo) >> 1)
        t = jax.lax.bitcast_convert_type(mid, jnp.float32)
        cnt = jnp.sum(jnp.where(d2 <= t, 1.0, 0.0), axis=1, keepdims=True)
        ge = cnt >= float(_K)
        return jnp.where(ge, lo, mid + 1), jnp.where(ge, mid, hi)

    lo, hi = jax.lax.fori_loop(0, 31, body, (lo, hi))
    r2k = jax.lax.bitcast_convert_type(hi, jnp.float32)

    inv_bw2 = 1.0 / jnp.maximum(r2k, 1e-16)
    dens_ref[...] = jnp.sum(jnp.exp(-d2 * inv_bw2), axis=1, keepdims=True)


def _peaks_kernel(feats_ref, qn_ref, dens_ref, out_ref):
    nb = dens_ref.shape[0]
    hw = dens_ref.shape[1]
    d3 = dens_ref[nb - 1:nb, :]
    iota = jax.lax.broadcasted_iota(jnp.int32, (1, hw), 1)

    vals = d3
    top_v = []
    centers = []
    for _ in range(_NCLUS):
        m = jnp.max(vals)
        idx = jnp.min(jnp.where(vals == m, iota, jnp.int32(2 ** 30)))
        top_v.append(jnp.reshape(m, (1, 1)))
        centers.append(feats_ref[pl.ds((nb - 1) * hw + idx, 1), :])
        vals = jnp.where(iota == idx, -jnp.inf, vals)

    tv = jnp.concatenate(top_v, axis=1)
    cen = jnp.concatenate(centers, axis=0)
    priors = tv / (jnp.sum(tv) + 1e-8)

    ones = jnp.ones((1, cen.shape[1]), jnp.float32)
    cn = jax.lax.dot_general(ones, cen * cen, (((1,), (1,)), ((), ())),
                             preferred_element_type=jnp.float32)
    prod = jax.lax.dot_general(feats_ref[...], cen, (((1,), (1,)), ((), ())),
                               preferred_element_type=jnp.float32)
    d2 = jnp.maximum(qn_ref[...] + cn - 2.0 * prod, 1e-12)
    dist = jnp.sqrt(d2)
    logits = -dist / _TEMP
    logits = logits - jnp.max(logits, axis=1, keepdims=True)
    e = jnp.exp(logits)
    soft = e / jnp.sum(e, axis=1, keepdims=True)
    out_ref[...] = jnp.sum(soft * priors, axis=1, keepdims=True)


def kernel(features, Wc, gamma, beta, memory_bank, perm):
    del perm
    b, c, h, w = features.shape
    n = b * h * w
    m_total = memory_bank.shape[0]
    n_tail = m_total - n

    flat = features.reshape(b, c, h * w).transpose(0, 2, 1).reshape(n, c)
    mtail = memory_bank[n:]

    feats, qn, mn = pl.pallas_call(
        _proj_kernel,
        out_shape=(
            jax.ShapeDtypeStruct((n, c), jnp.float32),
            jax.ShapeDtypeStruct((n, 1), jnp.float32),
            jax.ShapeDtypeStruct((1, m_total), jnp.float32),
        ),
    )(flat, Wc, gamma.reshape(1, c), beta.reshape(1, c), mtail)

    m_all = jnp.concatenate([feats, mtail], axis=0)

    grid = n // _TQ
    dens = pl.pallas_call(
        _density_kernel,
        grid=(grid,),
        in_specs=[
            pl.BlockSpec((_TQ, c), lambda i: (i, 0)),
            pl.BlockSpec((_TQ, 1), lambda i: (i, 0)),
            pl.BlockSpec((m_total, c), lambda i: (0, 0)),
            pl.BlockSpec((1, m_total), lambda i: (0, 0)),
        ],
        out_specs=pl.BlockSpec((_TQ, 1), lambda i: (i, 0)),
        out_shape=jax.ShapeDtypeStruct((n, 1), jnp.float32),
    )(feats, qn, m_all, mn)

    sem = pl.pallas_call(
        _peaks_kernel,
        out_shape=jax.ShapeDtypeStruct((n, 1), jnp.float32),
    )(feats, qn, dens.reshape(b, h * w))

    return sem.reshape(b, 1, h, w)
